# baseline (speedup 1.0000x reference)
import functools

import jax
import jax.numpy as jnp
from jax import lax
from jax.experimental import pallas as pl
from jax.experimental.pallas import tpu as pltpu
from jax.experimental.pallas import tpu_sc as plsc

N = 100000
D = 128
NG = 1024

TILE_N = 10000

NW = 32
ROWS_PER_W = N // NW
CHUNK = 125
NCHUNK = ROWS_PER_W // CHUNK
ROWS_PER_SUB = NG // 16


def _dot_nt(x, w):
    return lax.dot_general(x, w, (((1,), (1,)), ((), ())),
                           preferred_element_type=jnp.float32)


def _mlp_body(ns_ref, w1_ref, b1_ref, w2_ref, b2_ref, w3_ref, b3_ref, out_ref,
              z_ref):
    x = jnp.maximum(_dot_nt(ns_ref[...], w1_ref[...]) + b1_ref[...], 0.0)
    x = jnp.maximum(_dot_nt(x, w2_ref[...]) + b2_ref[...], 0.0)
    x = _dot_nt(x, w3_ref[...]) + b3_ref[...]
    out_ref[...] = x * (1.0 / (1.0 + jnp.exp(-x)))

    @pl.when(pl.program_id(0) == 0)
    def _():
        z_ref[...] = jnp.zeros_like(z_ref)


def _mlp(node_states, w1t, b1, w2t, b2, w3t, b3):
    grid = (N // TILE_N,)
    full = pl.BlockSpec((D, D), lambda i: (0, 0))
    bias = pl.BlockSpec((1, D), lambda i: (0, 0))
    return pl.pallas_call(
        _mlp_body,
        grid=grid,
        in_specs=[
            pl.BlockSpec((TILE_N, D), lambda i: (i, 0)),
            full, bias, full, bias, full, bias,
        ],

        out_specs=[pl.BlockSpec((TILE_N, D), lambda i: (i, 0)),
                   pl.BlockSpec((NG, D), lambda i: (0, 0))],
        out_shape=[jax.ShapeDtypeStruct((N, D), jnp.float32),
                   jax.ShapeDtypeStruct((NG, D), jnp.float32)],
        compiler_params=pltpu.CompilerParams(
            dimension_semantics=("parallel",)),
    )(node_states, w1t, b1, w2t, b2, w3t, b3)


def _sc_body(x_hbm, idx_hbm, zeros_hbm, out_hbm, xbuf0, xbuf1, xbuf2,
             idxs, obuf, acc, l0, l1, l2, s0a, s0b, s1a, s1b, s2a, s2b):
    c = lax.axis_index("c")
    s = lax.axis_index("s")
    w = c * 16 + s
    base = w * ROWS_PER_W
    cbase = w * NCHUNK

    bufs = (xbuf0, xbuf1, xbuf2)
    lsems = (l0, l1, l2)
    ssems = ((s0a, s0b), (s1a, s1b), (s2a, s2b))

    pltpu.sync_copy(idx_hbm.at[pl.ds(cbase, NCHUNK)], idxs)
    pltpu.sync_copy(zeros_hbm.at[pl.ds(s * ROWS_PER_SUB, ROWS_PER_SUB)],
                    acc.at[pl.ds(s * ROWS_PER_SUB, ROWS_PER_SUB)])
    plsc.subcore_barrier()

    def load(m, buf, sem):
        pltpu.async_copy(x_hbm.at[pl.ds(base + m * 2 * CHUNK, 2 * CHUNK)],
                         buf, sem)

    def wait_load(buf, sem):
        pltpu.make_async_copy(x_hbm.at[pl.ds(base, 2 * CHUNK)], buf,
                              sem).wait()

    def scatter(a, buf, half, sem):
        pltpu.async_copy(buf.at[pl.ds(half * CHUNK, CHUNK)],
                         acc.at[idxs.at[a]], sem, add=True)

    def wait_scatter(buf, sem):
        pltpu.make_async_copy(buf.at[pl.ds(0, CHUNK)], acc.at[idxs.at[0]],
                              sem).wait()

    for i in range(3):
        load(i, bufs[i], lsems[i])
    for i in range(3):
        wait_load(bufs[i], lsems[i])
        scatter(2 * i, bufs[i], 0, ssems[i][0])
        scatter(2 * i + 1, bufs[i], 1, ssems[i][1])

    def round_body(j, carry):
        m0 = 3 * j
        for i in range(3):
            wait_scatter(bufs[i], ssems[i][0])
            wait_scatter(bufs[i], ssems[i][1])
            load(m0 + i, bufs[i], lsems[i])
        for i in range(3):
            m = m0 + i
            wait_load(bufs[i], lsems[i])
            scatter(2 * m, bufs[i], 0, ssems[i][0])
            scatter(2 * m + 1, bufs[i], 1, ssems[i][1])
        return carry

    lax.fori_loop(1, 4, round_body, 0)
    wait_scatter(xbuf0, s0a)
    wait_scatter(xbuf0, s0b)
    pltpu.async_copy(x_hbm.at[pl.ds(base + (NCHUNK - 1) * CHUNK, CHUNK)],
                     xbuf0.at[pl.ds(0, CHUNK)], l0)
    pltpu.make_async_copy(x_hbm.at[pl.ds(base, CHUNK)],
                          xbuf0.at[pl.ds(0, CHUNK)], l0).wait()
    scatter(NCHUNK - 1, xbuf0, 0, s0a)
    wait_scatter(xbuf0, s0a)
    for i in (1, 2):
        wait_scatter(bufs[i], ssems[i][0])
        wait_scatter(bufs[i], ssems[i][1])
    plsc.subcore_barrier()

    pltpu.sync_copy(acc.at[pl.ds(s * ROWS_PER_SUB, ROWS_PER_SUB)], obuf)
    pltpu.sync_copy(obuf, out_hbm.at[c, pl.ds(s * ROWS_PER_SUB, ROWS_PER_SUB)])


def _sc_segment_sum(x, idx2d, zeros):
    mesh = plsc.VectorSubcoreMesh(core_axis_name="c", subcore_axis_name="s")
    fn = functools.partial(
        pl.kernel,
        out_type=jax.ShapeDtypeStruct((2, NG, D), jnp.float32),
        mesh=mesh,
        scratch_types=(
            [pltpu.VMEM((2 * CHUNK, D), jnp.float32)] * 3 + [
                pltpu.VMEM((NCHUNK, CHUNK), jnp.int32),
                pltpu.VMEM((ROWS_PER_SUB, D), jnp.float32),
                pltpu.VMEM_SHARED((NG, D), jnp.float32),
            ] + [pltpu.SemaphoreType.DMA] * 9
        ),
        compiler_params=pltpu.CompilerParams(use_tc_tiling_on_sc=False),
    )(_sc_body)
    return fn(x, idx2d, zeros)


def _combine_body(p_ref, o_ref):
    o_ref[...] = p_ref[0] + p_ref[1]


def _combine(partials):
    return pl.pallas_call(
        _combine_body,
        out_shape=jax.ShapeDtypeStruct((NG, D), jnp.float32),
    )(partials)


def kernel(node_states, graph_idx, W1, b1, W2, b2, W3, b3):
    idx2d = graph_idx.astype(jnp.int32).reshape(N // CHUNK, CHUNK)
    x, zeros = _mlp(node_states, W1, b1.reshape(1, D), W2, b2.reshape(1, D),
                    W3, b3.reshape(1, D))
    partials = _sc_segment_sum(x, idx2d, zeros)
    return _combine(partials)


# speedup vs baseline: 1.0128x; 1.0128x over previous
import functools

import jax
import jax.numpy as jnp
from jax import lax
from jax.experimental import pallas as pl
from jax.experimental.pallas import tpu as pltpu
from jax.experimental.pallas import tpu_sc as plsc

N = 100000
D = 128
NG = 1024

TILE_N = 10000

NW = 32
ROWS_PER_W = N // NW
CHUNK = 125
NCHUNK = ROWS_PER_W // CHUNK
ROWS_PER_SUB = NG // 16


def _dot_nt(x, w):
    return lax.dot_general(x, w, (((1,), (1,)), ((), ())),
                           preferred_element_type=jnp.float32)


def _mlp_body(ns_ref, w1_ref, b1_ref, w2_ref, b2_ref, w3_ref, b3_ref, out_ref,
              z_ref):
    x = jnp.maximum(_dot_nt(ns_ref[...], w1_ref[...]) + b1_ref[...], 0.0)
    x = jnp.maximum(_dot_nt(x, w2_ref[...]) + b2_ref[...], 0.0)
    x = _dot_nt(x, w3_ref[...]) + b3_ref[...]
    out_ref[...] = x * (1.0 / (1.0 + jnp.exp(-x)))

    @pl.when(pl.program_id(0) == 0)
    def _():
        z_ref[...] = jnp.zeros_like(z_ref)


def _mlp(node_states, w1t, b1, w2t, b2, w3t, b3):
    grid = (N // TILE_N,)
    full = pl.BlockSpec((D, D), lambda i: (0, 0))
    bias = pl.BlockSpec((1, D), lambda i: (0, 0))
    return pl.pallas_call(
        _mlp_body,
        grid=grid,
        in_specs=[
            pl.BlockSpec((TILE_N, D), lambda i: (i, 0)),
            full, bias, full, bias, full, bias,
        ],

        out_specs=[pl.BlockSpec((TILE_N, D), lambda i: (i, 0)),
                   pl.BlockSpec((NG, D), lambda i: (0, 0))],
        out_shape=[jax.ShapeDtypeStruct((N, D), jnp.float32),
                   jax.ShapeDtypeStruct((NG, D), jnp.float32)],
        compiler_params=pltpu.CompilerParams(
            dimension_semantics=("parallel",)),
    )(node_states, w1t, b1, w2t, b2, w3t, b3)


def _sc_body(x_hbm, idx_hbm, zeros_hbm, out_hbm, xbuf0, xbuf1, xbuf2, xbuf3,
             xbuf4, xbuf5, idxs, obuf, acc, l0, l1, l2, l3, l4, l5,
             t0, t1, t2, t3, t4, t5):
    c = lax.axis_index("c")
    s = lax.axis_index("s")
    w = c * 16 + s
    base = w * ROWS_PER_W
    cbase = w * NCHUNK

    bufs = (xbuf0, xbuf1, xbuf2, xbuf3, xbuf4, xbuf5)
    lsems = (l0, l1, l2, l3, l4, l5)
    tsems = (t0, t1, t2, t3, t4, t5)

    pltpu.sync_copy(idx_hbm.at[pl.ds(cbase, NCHUNK)], idxs)
    pltpu.sync_copy(zeros_hbm.at[pl.ds(s * ROWS_PER_SUB, ROWS_PER_SUB)],
                    acc.at[pl.ds(s * ROWS_PER_SUB, ROWS_PER_SUB)])
    plsc.subcore_barrier()

    def load(k, buf, sem):
        pltpu.async_copy(x_hbm.at[pl.ds(base + k * CHUNK, CHUNK)], buf, sem)

    def wait_load(buf, sem):
        pltpu.make_async_copy(x_hbm.at[pl.ds(base, CHUNK)], buf, sem).wait()

    def scatter(a, buf, sem):
        pltpu.async_copy(buf, acc.at[idxs.at[a]], sem, add=True)

    def wait_scatter(buf, sem):
        pltpu.make_async_copy(buf, acc.at[idxs.at[0]], sem).wait()

    for i in range(6):
        load(i, bufs[i], lsems[i])
    for i in range(6):
        wait_load(bufs[i], lsems[i])
        scatter(i, bufs[i], tsems[i])

    def round_body(j, carry):
        a = 6 * j
        for i in range(6):
            wait_scatter(bufs[i], tsems[i])
            load(a + i, bufs[i], lsems[i])
        for i in range(6):
            wait_load(bufs[i], lsems[i])
            scatter(a + i, bufs[i], tsems[i])
        return carry

    lax.fori_loop(1, (NCHUNK - 1) // 6, round_body, 0)
    wait_scatter(xbuf0, t0)
    load(NCHUNK - 1, xbuf0, l0)
    wait_load(xbuf0, l0)
    scatter(NCHUNK - 1, xbuf0, t0)
    for i in range(6):
        wait_scatter(bufs[i], tsems[i])
    plsc.subcore_barrier()

    pltpu.sync_copy(acc.at[pl.ds(s * ROWS_PER_SUB, ROWS_PER_SUB)], obuf)
    pltpu.sync_copy(obuf, out_hbm.at[c, pl.ds(s * ROWS_PER_SUB, ROWS_PER_SUB)])


def _sc_segment_sum(x, idx2d, zeros):
    mesh = plsc.VectorSubcoreMesh(core_axis_name="c", subcore_axis_name="s")
    fn = functools.partial(
        pl.kernel,
        out_type=jax.ShapeDtypeStruct((2, NG, D), jnp.float32),
        mesh=mesh,
        scratch_types=(
            [pltpu.VMEM((CHUNK, D), jnp.float32)] * 6 + [
                pltpu.VMEM((NCHUNK, CHUNK), jnp.int32),
                pltpu.VMEM((ROWS_PER_SUB, D), jnp.float32),
                pltpu.VMEM_SHARED((NG, D), jnp.float32),
            ] + [pltpu.SemaphoreType.DMA] * 12
        ),
        compiler_params=pltpu.CompilerParams(use_tc_tiling_on_sc=False),
    )(_sc_body)
    return fn(x, idx2d, zeros)


def _combine_body(p_ref, o_ref):
    o_ref[...] = p_ref[0] + p_ref[1]


def _combine(partials):
    return pl.pallas_call(
        _combine_body,
        out_shape=jax.ShapeDtypeStruct((NG, D), jnp.float32),
    )(partials)


def kernel(node_states, graph_idx, W1, b1, W2, b2, W3, b3):
    idx2d = graph_idx.astype(jnp.int32).reshape(N // CHUNK, CHUNK)
    x, zeros = _mlp(node_states, W1, b1.reshape(1, D), W2, b2.reshape(1, D),
                    W3, b3.reshape(1, D))
    partials = _sc_segment_sum(x, idx2d, zeros)
    return _combine(partials)
